# baseline (speedup 1.0000x reference)
import functools
import numpy as np
import jax
import jax.numpy as jnp
from jax import lax
from jax.experimental import pallas as pl
from jax.experimental.pallas import tpu as pltpu
from jax.experimental.pallas import tpu_sc as plsc

N = 10000
D = 128
H = 64
ED = 4
L = 4
OUT = 4

NC, NS = 2, 16
NW = NC * NS
SCB = 128
EB = 4096
NBLK = 2000
N_PAD = 10240
RPS = N_PAD // NS

f32 = jnp.float32
bf16 = jnp.bfloat16


def _ln(x):
    m = jnp.mean(x, axis=-1, keepdims=True)
    v = jnp.mean((x - m) ** 2, axis=-1, keepdims=True)
    return (x - m) / jnp.sqrt(v + 1e-5)


def _leaky(x):
    return jnp.where(x >= 0, x, 0.1 * x)


def _gelu(x):
    return 0.5 * x * (1.0 + lax.erf(x / jnp.sqrt(jnp.float32(2.0))))


def _act(x, kind):
    return _gelu(x) if kind == 'gelu' else _leaky(x)


def _sc_mesh():
    return plsc.VectorSubcoreMesh(
        core_axis_name="c", subcore_axis_name="s",
        num_cores=NC, num_subcores=NS)


def _sc_gather(h2, src2d, dst2d, with_isf):
    nb = src2d.shape[0] // NW
    e_pad = NW * nb * SCB
    npairs = nb // 2

    def body(h_hbm, src_hbm, dst_hbm, *refs):
        if with_isf:
            (xc_hbm, isf_hbm, idx_d, idx_s, ri_a, rj_a, ri_b, rj_b,
             isf_v, hs, sem_ga, sem_gb, sem_wa, sem_wb) = refs
        else:
            (xc_hbm, idx_d, idx_s, ri_a, rj_a, ri_b, rj_b,
             hs, sem_ga, sem_gb, sem_wa, sem_wb) = refs
        c = lax.axis_index("c")
        s = lax.axis_index("s")
        wid = s * NC + c
        pltpu.sync_copy(h_hbm.at[pl.ds(s * RPS, RPS), pl.ds(0, H)],
                        hs.at[pl.ds(s * RPS, RPS)])
        pltpu.sync_copy(dst_hbm.at[pl.ds(wid * nb, nb)], idx_d)
        pltpu.sync_copy(src_hbm.at[pl.ds(wid * nb, nb)], idx_s)
        plsc.subcore_barrier()

        if with_isf:
            def isf_step(j, carry):
                for k in range(SCB // 16):
                    d16 = idx_d[j, pl.ds(k * 16, 16)]
                    s16 = idx_s[j, pl.ds(k * 16, 16)]
                    isf_v[j, pl.ds(k * 16, 16)] = jnp.where(
                        d16 == s16, f32(1.0), f32(0.0))
                return carry
            lax.fori_loop(0, nb, isf_step, 0)
            pltpu.sync_copy(
                isf_v, isf_hbm.at[pl.ds(wid * nb, nb)])

        base0 = wid * nb * SCB

        def gather(j, ri, rj, sem):
            pltpu.async_copy(hs.at[idx_d.at[j]], ri, sem)
            pltpu.async_copy(hs.at[idx_s.at[j]], rj, sem)

        def drain2(sem):
            pltpu.make_async_copy(
                xc_hbm.at[pl.ds(0, SCB), pl.ds(0, H)], ri_a, sem).wait()
            pltpu.make_async_copy(
                xc_hbm.at[pl.ds(0, SCB), pl.ds(0, H)], ri_a, sem).wait()

        def write(j, ri, rj, sem):
            base = base0 + j * SCB
            pltpu.async_copy(ri, xc_hbm.at[pl.ds(base, SCB), pl.ds(0, H)], sem)
            pltpu.async_copy(rj, xc_hbm.at[pl.ds(base, SCB), pl.ds(H, H)], sem)

        gather(0, ri_a, rj_a, sem_ga)

        def step(jj, carry):
            j0 = 2 * jj
            j1 = 2 * jj + 1

            @pl.when(jj > 0)
            def _():
                drain2(sem_wb)
            gather(j1, ri_b, rj_b, sem_gb)
            drain2(sem_ga)
            write(j0, ri_a, rj_a, sem_wa)
            drain2(sem_wa)

            @pl.when(jj + 1 < npairs)
            def _():
                gather(j0 + 2, ri_a, rj_a, sem_ga)
            drain2(sem_gb)
            write(j1, ri_b, rj_b, sem_wb)
            return carry

        lax.fori_loop(0, npairs, step, 0)
        drain2(sem_wb)

    out_type = [
        jax.ShapeDtypeStruct((e_pad, 2 * H), bf16),
    ]
    scratch = [
        pltpu.VMEM((nb, SCB), jnp.int32),
        pltpu.VMEM((nb, SCB), jnp.int32),
        pltpu.VMEM((SCB, H), bf16),
        pltpu.VMEM((SCB, H), bf16),
        pltpu.VMEM((SCB, H), bf16),
        pltpu.VMEM((SCB, H), bf16),
    ]
    if with_isf:
        out_type.append(jax.ShapeDtypeStruct((NW * nb, SCB), f32))
        scratch.append(pltpu.VMEM((nb, SCB), f32))
    scratch.append(pltpu.VMEM_SHARED((N_PAD, H), bf16))
    scratch += [pltpu.SemaphoreType.DMA] * 4

    return pl.kernel(
        body,
        out_type=tuple(out_type) if with_isf else out_type[0],
        mesh=_sc_mesh(),
        scratch_types=scratch,
        compiler_params=pltpu.CompilerParams(use_tc_tiling_on_sc=False),
    )(h2, src2d, dst2d)


def _sc_scatter(msg, dst2d, zeros_pad):
    nb = dst2d.shape[0] // NW
    npairs = nb // 2

    def body(msg_hbm, dst_hbm, z_hbm, out_hbm, idx_v, msg_a, msg_b, acc,
             sem_ma, sem_mb, sem_sa, sem_sb):
        c = lax.axis_index("c")
        s = lax.axis_index("s")
        wid = s * NC + c
        pltpu.sync_copy(z_hbm.at[pl.ds(s * RPS, RPS)],
                        acc.at[pl.ds(s * RPS, RPS)])
        pltpu.sync_copy(dst_hbm.at[pl.ds(wid * nb, nb)], idx_v)
        plsc.subcore_barrier()
        base0 = wid * nb * SCB

        def drain1(buf, sem):
            pltpu.make_async_copy(
                msg_hbm.at[pl.ds(0, SCB), pl.ds(0, H)], buf, sem).wait()

        pltpu.async_copy(
            msg_hbm.at[pl.ds(base0, SCB), pl.ds(0, H)], msg_a, sem_ma)

        def step(jj, carry):
            j0 = 2 * jj
            j1 = 2 * jj + 1

            @pl.when(jj > 0)
            def _():
                drain1(msg_b, sem_sb)
            pltpu.async_copy(
                msg_hbm.at[pl.ds(base0 + j1 * SCB, SCB), pl.ds(0, H)],
                msg_b, sem_mb)
            drain1(msg_a, sem_ma)
            pltpu.async_copy(msg_a, acc.at[idx_v.at[j0]], sem_sa, add=True)
            drain1(msg_a, sem_sa)

            @pl.when(jj + 1 < npairs)
            def _():
                pltpu.async_copy(
                    msg_hbm.at[pl.ds(base0 + (j0 + 2) * SCB, SCB),
                               pl.ds(0, H)],
                    msg_a, sem_ma)
            drain1(msg_b, sem_mb)
            pltpu.async_copy(msg_b, acc.at[idx_v.at[j1]], sem_sb, add=True)
            return carry

        lax.fori_loop(0, npairs, step, 0)
        drain1(msg_b, sem_sb)
        plsc.subcore_barrier()
        pltpu.sync_copy(acc.at[pl.ds(s * RPS, RPS)],
                        out_hbm.at[c, pl.ds(s * RPS, RPS)])

    return pl.kernel(
        body,
        out_type=jax.ShapeDtypeStruct((NC, N_PAD, H), f32),
        mesh=_sc_mesh(),
        scratch_types=[
            pltpu.VMEM((nb, SCB), jnp.int32),
            pltpu.VMEM((SCB, H), f32),
            pltpu.VMEM((SCB, H), f32),
            pltpu.VMEM_SHARED((N_PAD, H), f32),
            pltpu.SemaphoreType.DMA,
            pltpu.SemaphoreType.DMA,
            pltpu.SemaphoreType.DMA,
            pltpu.SemaphoreType.DMA,
        ],
        compiler_params=pltpu.CompilerParams(use_tc_tiling_on_sc=False),
    )(msg, dst2d, zeros_pad)


def _full(shape):
    return pl.BlockSpec(shape, lambda i: (0,) * len(shape))


def _emb_call(x, wl, bl, wp, bp, wc1, wc2, bc):
    def body(x_ref, wl_r, bl_r, wp_r, bp_r, wc1_r, wc2_r, bc_r,
             h2_ref, hf_ref):
        xv = x_ref[...]
        lin = xv @ wl_r[...] + bl_r[...]
        pw = (xv * xv) @ wp_r[...] + bp_r[...]
        hv = lin @ wc1_r[...] + pw @ wc2_r[...] + bc_r[...]
        h2_ref[:, :H] = hv.astype(bf16)
        h2_ref[:, H:] = jnp.zeros((NBLK, H), bf16)
        hf_ref[...] = hv

    return pl.pallas_call(
        body,
        grid=(N // NBLK,),
        in_specs=[
            pl.BlockSpec((NBLK, D), lambda i: (i, 0)),
            _full((D, H)), _full((1, H)),
            _full((D, H)), _full((1, H)),
            _full((H, H)), _full((H, H)), _full((1, H)),
        ],
        out_specs=[
            pl.BlockSpec((NBLK, 2 * H), lambda i: (i, 0)),
            pl.BlockSpec((NBLK, H), lambda i: (i, 0)),
        ],
        out_shape=[
            jax.ShapeDtypeStruct((N_PAD, 2 * H), bf16),
            jax.ShapeDtypeStruct((N, H), f32),
        ],
    )(x, wl, bl, wp, bp, wc1, wc2, bc)


def _edge_call(kind, xc, ea, isf, wc, we, b1, mb, wd, b2, ad, adb, sf):
    e_pad = xc.shape[0]

    def body(xc_r, ea_r, isf_r, wc_r, we_r, b1_r, mb_r,
             wd_r, b2_r, ad_r, adb_r, sf_r, out_ref):
        z = (jnp.dot(xc_r[...], wc_r[...], preferred_element_type=f32)
             + ea_r[...] @ we_r[...] + b1_r[...])
        if kind == 'gelu':
            a = _gelu(z)
        else:
            lane = lax.broadcasted_iota(jnp.int32, (EB, 2 * H), 1)
            a = jnp.where(lane < H, _leaky(z), _gelu(z))
        mu = a @ mb_r[...]
        d = a - mu
        var = (d * d) @ mb_r[...]
        ln = d * lax.rsqrt(var + 1e-5)
        m12 = ln @ wd_r[...] + b2_r[...]
        dl = m12 @ ad_r[...] + adb_r[...]
        aw0 = jax.nn.sigmoid(dl)
        m1 = m12[:, :H]
        m2 = m12[:, H:]
        msg = m2 + aw0 * (m1 - m2)
        isfv = isf_r[...]
        scale = isfv + (1.0 - isfv) * sf_r[0, 0]
        out_ref[:, :H] = msg * scale
        out_ref[:, H:] = jnp.zeros((EB, H), f32)

    return pl.pallas_call(
        body,
        grid=(e_pad // EB,),
        in_specs=[
            pl.BlockSpec((EB, 2 * H), lambda i: (i, 0)),
            pl.BlockSpec((EB, ED), lambda i: (i, 0)),
            pl.BlockSpec((EB, 1), lambda i: (i, 0)),
            _full((2 * H, 2 * H)), _full((ED, 2 * H)),
            _full((1, 2 * H)), _full((2 * H, 2 * H)), _full((2 * H, 2 * H)),
            _full((1, 2 * H)), _full((2 * H, 1)), _full((1, 1)),
            _full((1, 1)),
        ],
        out_specs=pl.BlockSpec((EB, 2 * H), lambda i: (i, 0)),
        out_shape=jax.ShapeDtypeStruct((e_pad, 2 * H), f32),
    )(xc, ea, isf, wc, we, b1, mb, wd, b2, ad, adb, sf)


def _upd_call(kind, resid, ag, hf, acc, gwa, gwh, gb, u1a, u1h, u1b,
              u2w, u2b, attn_i):
    def body(ag_r, h_ref, acc_r, gwa_r, gwh_r, gb_r, u1a_r, u1h_r, u1b_r,
             u2w_r, u2b_r, at_r, hout_ref, hf_ref, accout_ref, hsum_ref):
        a = ag_r[0] + ag_r[1]
        hv = h_ref[...]
        gate = jax.nn.sigmoid(a @ gwa_r[...] + hv @ gwh_r[...] + gb_r[...])
        u1 = _ln(_act(a @ u1a_r[...] + hv @ u1h_r[...] + u1b_r[...], kind))
        u2 = hv + _act(u1 @ u2w_r[...] + u2b_r[...], kind)
        hn = _ln(hv * (1.0 - gate) + u2 * gate)
        hout = hv + hn if resid else hn
        accout = acc_r[...] + at_r[0, 0] * hout
        hout_ref[:, :H] = hout.astype(bf16)
        hout_ref[:, H:] = jnp.zeros((NBLK, H), bf16)
        hf_ref[...] = hout
        accout_ref[...] = accout

        @pl.when(pl.program_id(0) == 0)
        def _init():
            hsum_ref[...] = jnp.zeros_like(hsum_ref)

        hsum_ref[...] += jnp.sum(accout, axis=0, keepdims=True)

    return pl.pallas_call(
        body,
        grid=(N // NBLK,),
        in_specs=[
            pl.BlockSpec((NC, NBLK, H), lambda i: (0, i, 0)),
            pl.BlockSpec((NBLK, H), lambda i: (i, 0)),
            pl.BlockSpec((NBLK, H), lambda i: (i, 0)),
            _full((H, H)), _full((H, H)), _full((1, H)),
            _full((H, H)), _full((H, H)), _full((1, H)),
            _full((H, H)), _full((1, H)), _full((1, 1)),
        ],
        out_specs=[
            pl.BlockSpec((NBLK, 2 * H), lambda i: (i, 0)),
            pl.BlockSpec((NBLK, H), lambda i: (i, 0)),
            pl.BlockSpec((NBLK, H), lambda i: (i, 0)),
            pl.BlockSpec((1, H), lambda i: (0, 0)),
        ],
        out_shape=[
            jax.ShapeDtypeStruct((N_PAD, 2 * H), bf16),
            jax.ShapeDtypeStruct((N, H), f32),
            jax.ShapeDtypeStruct((N, H), f32),
            jax.ShapeDtypeStruct((1, H), f32),
        ],
    )(ag, hf, acc, gwa, gwh, gb, u1a, u1h, u1b, u2w, u2b, attn_i)


def _head_call(hsum, w1, b1, w2, b2, w3, b3):
    def body(hs_r, w1_r, b1_r, w2_r, b2_r, w3_r, b3_r, out_ref):
        g = hs_r[...] * f32(1.0 / N)
        g = _ln(_leaky(g @ w1_r[...] + b1_r[...]))
        g = _leaky(g @ w2_r[...] + b2_r[...])
        out_ref[...] = g @ w3_r[...] + b3_r[...]

    return pl.pallas_call(
        body,
        grid=(1,),
        in_specs=[
            _full((1, H)),
            _full((H, H)), _full((1, H)),
            _full((H, H // 2)), _full((1, H // 2)),
            _full((H // 2, OUT)), _full((1, OUT)),
        ],
        out_specs=pl.BlockSpec((1, OUT), lambda i: (0, 0)),
        out_shape=jax.ShapeDtypeStruct((1, OUT), f32),
    )(hsum, w1, b1, w2, b2, w3, b3)


_MB = np.kron(np.eye(2, dtype=np.float32),
              np.full((H, H), 1.0 / H, np.float32))


def kernel(x, edge_index, edge_attr, params):
    p = params
    E = edge_index.shape[1]
    e_tot = E + N
    chunk = NW * SCB * 2
    e_pad = ((e_tot + chunk - 1) // chunk) * chunk
    padlen = e_pad - e_tot

    sl = jnp.arange(N, dtype=jnp.int32)
    src = jnp.concatenate([edge_index[0].astype(jnp.int32), sl])
    dst = jnp.concatenate([edge_index[1].astype(jnp.int32), sl])
    zpad = jnp.zeros((padlen,), jnp.int32)
    src2d = jnp.concatenate([src, zpad]).reshape(-1, SCB)
    dst2d = jnp.concatenate([dst, zpad]).reshape(-1, SCB)
    dst2d_s = jnp.concatenate(
        [dst, jnp.full((padlen,), N, jnp.int32)]).reshape(-1, SCB)

    dummy = jnp.zeros((N, ED), f32).at[:, 0].set(1.0)
    ea = jnp.concatenate(
        [edge_attr.astype(f32), dummy, jnp.zeros((padlen, ED), f32)], axis=0)

    def row(b):
        return b.reshape(1, -1).astype(f32)

    h2b, hf = _emb_call(
        x.astype(f32),
        p['emb_lin_w'].T.astype(f32), row(p['emb_lin_b']),
        p['emb_pow_w'].T.astype(f32), row(p['emb_pow_b']),
        p['emb_comb_w'][:, :H].T.astype(f32),
        p['emb_comb_w'][:, H:].T.astype(f32), row(p['emb_comb_b']),
    )

    attn = jax.nn.softmax(p['layer_attn'].astype(f32))
    acc = jnp.zeros((N, H), f32)
    zeros_pad = jnp.zeros((N_PAD, H), f32)
    mb = jnp.asarray(_MB)
    zhh = jnp.zeros((H, H), f32)
    hsum = None
    isf = None

    for i in range(L):
        kind = 'gelu' if i % 2 == 1 else 'leaky'
        if i == 0:
            xc, isfw = _sc_gather(h2b, src2d, dst2d, True)
            isf = isfw.reshape(e_pad, 1)
        else:
            xc = _sc_gather(h2b, src2d, dst2d, False)
        w11 = p['mp1_w1'][i].astype(f32)
        w12 = p['mp2_w1'][i].astype(f32)
        wd = jnp.concatenate([
            jnp.concatenate([p['mp1_w2'][i].T.astype(f32), zhh], axis=1),
            jnp.concatenate([zhh, p['mp2_w2'][i].T.astype(f32)], axis=1),
        ], axis=0)
        aw_ = p['attn_w'][i].astype(f32)
        ab_ = p['attn_b'][i].astype(f32)
        wc = jnp.concatenate([
            jnp.concatenate([w11[:, :H].T, w12[:, :H].T], axis=1),
            jnp.concatenate([w11[:, H:2 * H].T, w12[:, H:2 * H].T], axis=1),
        ], axis=0)
        msg = _edge_call(
            kind, xc, ea, isf, wc.astype(bf16),
            jnp.concatenate([w11[:, 2 * H:].T, w12[:, 2 * H:].T], axis=1),
            jnp.concatenate(
                [row(p['mp1_b1'][i]), row(p['mp2_b1'][i])], axis=1),
            mb, wd,
            jnp.concatenate(
                [row(p['mp1_b2'][i]), row(p['mp2_b2'][i])], axis=1),
            (aw_[0] - aw_[1]).reshape(2 * H, 1),
            (ab_[0] - ab_[1]).reshape(1, 1),
            p['scale_factor'][i].reshape(1, 1).astype(f32),
        )
        ag = _sc_scatter(msg, dst2d_s, zeros_pad)
        gw = p['gate_w'][i].astype(f32)
        u1w = p['upd1_w'][i].astype(f32)
        h2b, hf, acc, hsum = _upd_call(
            kind, i % 2 == 1, ag, hf, acc,
            gw[:, :H].T, gw[:, H:].T, row(p['gate_b'][i]),
            u1w[:, :H].T, u1w[:, H:].T, row(p['upd1_b'][i]),
            p['upd2_w'][i].T.astype(f32), row(p['upd2_b'][i]),
            attn[i].reshape(1, 1),
        )

    return _head_call(
        hsum,
        p['pre_w1'].T.astype(f32), row(p['pre_b1']),
        p['pre_w2'].T.astype(f32), row(p['pre_b2']),
        p['out_w'].T.astype(f32), row(p['out_b']),
    )


# speedup vs baseline: 1.5899x; 1.5899x over previous
import functools
import numpy as np
import jax
import jax.numpy as jnp
from jax import lax
from jax.experimental import pallas as pl
from jax.experimental.pallas import tpu as pltpu
from jax.experimental.pallas import tpu_sc as plsc

N = 10000
D = 128
H = 64
ED = 4
L = 4
OUT = 4

NC, NS = 2, 16
NW = NC * NS
SCB = 128
EB = 4096
NBLK = 2000
N_PAD = 10240
RPS = N_PAD // NS

f32 = jnp.float32
bf16 = jnp.bfloat16


def _ln(x):
    m = jnp.mean(x, axis=-1, keepdims=True)
    v = jnp.mean((x - m) ** 2, axis=-1, keepdims=True)
    return (x - m) / jnp.sqrt(v + 1e-5)


def _leaky(x):
    return jnp.where(x >= 0, x, 0.1 * x)


def _gelu(x):
    return 0.5 * x * (1.0 + lax.erf(x / jnp.sqrt(jnp.float32(2.0))))


def _act(x, kind):
    return _gelu(x) if kind == 'gelu' else _leaky(x)


def _sc_mesh():
    return plsc.VectorSubcoreMesh(
        core_axis_name="c", subcore_axis_name="s",
        num_cores=NC, num_subcores=NS)


def _sc_gather(h2, src2d, dst2d, with_isf):
    nb = src2d.shape[0] // NW
    e_pad = NW * nb * SCB
    npairs = nb // 2

    def body(h_hbm, src_hbm, dst_hbm, *refs):
        if with_isf:
            (xc_hbm, isf_hbm, idx_d, idx_s, ri_a, rj_a, ri_b, rj_b,
             isf_v, hs, sem_ga, sem_gb, sem_wa, sem_wb) = refs
        else:
            (xc_hbm, idx_d, idx_s, ri_a, rj_a, ri_b, rj_b,
             hs, sem_ga, sem_gb, sem_wa, sem_wb) = refs
        c = lax.axis_index("c")
        s = lax.axis_index("s")
        wid = s * NC + c
        pltpu.sync_copy(h_hbm.at[pl.ds(s * RPS, RPS), pl.ds(0, H)],
                        hs.at[pl.ds(s * RPS, RPS)])
        pltpu.sync_copy(dst_hbm.at[pl.ds(wid * nb, nb)], idx_d)
        pltpu.sync_copy(src_hbm.at[pl.ds(wid * nb, nb)], idx_s)
        plsc.subcore_barrier()

        if with_isf:
            def isf_step(j, carry):
                for k in range(SCB // 16):
                    d16 = idx_d[j, pl.ds(k * 16, 16)]
                    s16 = idx_s[j, pl.ds(k * 16, 16)]
                    isf_v[j, pl.ds(k * 16, 16)] = jnp.where(
                        d16 == s16, f32(1.0), f32(0.0))
                return carry
            lax.fori_loop(0, nb, isf_step, 0)
            pltpu.sync_copy(
                isf_v, isf_hbm.at[pl.ds(wid * nb, nb)])

        base0 = wid * nb * SCB

        def gather(j, ri, rj, sem):
            pltpu.async_copy(hs.at[idx_d.at[j]], ri, sem)
            pltpu.async_copy(hs.at[idx_s.at[j]], rj, sem)

        def drain2(sem):
            pltpu.make_async_copy(
                xc_hbm.at[pl.ds(0, SCB), pl.ds(0, H)], ri_a, sem).wait()
            pltpu.make_async_copy(
                xc_hbm.at[pl.ds(0, SCB), pl.ds(0, H)], ri_a, sem).wait()

        def write(j, ri, rj, sem):
            base = base0 + j * SCB
            pltpu.async_copy(ri, xc_hbm.at[pl.ds(base, SCB), pl.ds(0, H)], sem)
            pltpu.async_copy(rj, xc_hbm.at[pl.ds(base, SCB), pl.ds(H, H)], sem)

        gather(0, ri_a, rj_a, sem_ga)

        def step(jj, carry):
            j0 = 2 * jj
            j1 = 2 * jj + 1

            @pl.when(jj > 0)
            def _():
                drain2(sem_wb)
            gather(j1, ri_b, rj_b, sem_gb)
            drain2(sem_ga)
            write(j0, ri_a, rj_a, sem_wa)
            drain2(sem_wa)

            @pl.when(jj + 1 < npairs)
            def _():
                gather(j0 + 2, ri_a, rj_a, sem_ga)
            drain2(sem_gb)
            write(j1, ri_b, rj_b, sem_wb)
            return carry

        lax.fori_loop(0, npairs, step, 0)
        drain2(sem_wb)

    out_type = [
        jax.ShapeDtypeStruct((e_pad, 2 * H), f32),
    ]
    scratch = [
        pltpu.VMEM((nb, SCB), jnp.int32),
        pltpu.VMEM((nb, SCB), jnp.int32),
        pltpu.VMEM((SCB, H), f32),
        pltpu.VMEM((SCB, H), f32),
        pltpu.VMEM((SCB, H), f32),
        pltpu.VMEM((SCB, H), f32),
    ]
    if with_isf:
        out_type.append(jax.ShapeDtypeStruct((NW * nb, SCB), f32))
        scratch.append(pltpu.VMEM((nb, SCB), f32))
    scratch.append(pltpu.VMEM_SHARED((N_PAD, H), f32))
    scratch += [pltpu.SemaphoreType.DMA] * 4

    return pl.kernel(
        body,
        out_type=tuple(out_type) if with_isf else out_type[0],
        mesh=_sc_mesh(),
        scratch_types=scratch,
        compiler_params=pltpu.CompilerParams(use_tc_tiling_on_sc=False),
    )(h2, src2d, dst2d)


def _sc_scatter(msg, dst2d, zeros_pad):
    nb = dst2d.shape[0] // NW
    npairs = nb // 2

    def body(msg_hbm, dst_hbm, z_hbm, out_hbm, idx_v, msg_a, msg_b, acc,
             sem_ma, sem_mb, sem_sa, sem_sb):
        c = lax.axis_index("c")
        s = lax.axis_index("s")
        wid = s * NC + c
        pltpu.sync_copy(z_hbm.at[pl.ds(s * RPS, RPS)],
                        acc.at[pl.ds(s * RPS, RPS)])
        pltpu.sync_copy(dst_hbm.at[pl.ds(wid * nb, nb)], idx_v)
        plsc.subcore_barrier()
        base0 = wid * nb * SCB

        def drain1(buf, sem):
            pltpu.make_async_copy(
                msg_hbm.at[pl.ds(0, SCB), pl.ds(0, H)], buf, sem).wait()

        pltpu.async_copy(
            msg_hbm.at[pl.ds(base0, SCB), pl.ds(0, H)], msg_a, sem_ma)

        def step(jj, carry):
            j0 = 2 * jj
            j1 = 2 * jj + 1

            @pl.when(jj > 0)
            def _():
                drain1(msg_b, sem_sb)
            pltpu.async_copy(
                msg_hbm.at[pl.ds(base0 + j1 * SCB, SCB), pl.ds(0, H)],
                msg_b, sem_mb)
            drain1(msg_a, sem_ma)
            pltpu.async_copy(msg_a, acc.at[idx_v.at[j0]], sem_sa, add=True)
            drain1(msg_a, sem_sa)

            @pl.when(jj + 1 < npairs)
            def _():
                pltpu.async_copy(
                    msg_hbm.at[pl.ds(base0 + (j0 + 2) * SCB, SCB),
                               pl.ds(0, H)],
                    msg_a, sem_ma)
            drain1(msg_b, sem_mb)
            pltpu.async_copy(msg_b, acc.at[idx_v.at[j1]], sem_sb, add=True)
            return carry

        lax.fori_loop(0, npairs, step, 0)
        drain1(msg_b, sem_sb)
        plsc.subcore_barrier()
        pltpu.sync_copy(acc.at[pl.ds(s * RPS, RPS)],
                        out_hbm.at[c, pl.ds(s * RPS, RPS)])

    return pl.kernel(
        body,
        out_type=jax.ShapeDtypeStruct((NC, N_PAD, H), f32),
        mesh=_sc_mesh(),
        scratch_types=[
            pltpu.VMEM((nb, SCB), jnp.int32),
            pltpu.VMEM((SCB, H), f32),
            pltpu.VMEM((SCB, H), f32),
            pltpu.VMEM_SHARED((N_PAD, H), f32),
            pltpu.SemaphoreType.DMA,
            pltpu.SemaphoreType.DMA,
            pltpu.SemaphoreType.DMA,
            pltpu.SemaphoreType.DMA,
        ],
        compiler_params=pltpu.CompilerParams(use_tc_tiling_on_sc=False),
    )(msg, dst2d, zeros_pad)


def _full(shape):
    return pl.BlockSpec(shape, lambda i: (0,) * len(shape))


def _emb_call(x, wl, bl, wp, bp, wc1, wc2, bc):
    def body(x_ref, wl_r, bl_r, wp_r, bp_r, wc1_r, wc2_r, bc_r, h2_ref):
        xv = x_ref[...]
        lin = xv @ wl_r[...] + bl_r[...]
        pw = (xv * xv) @ wp_r[...] + bp_r[...]
        h2_ref[:, :H] = lin @ wc1_r[...] + pw @ wc2_r[...] + bc_r[...]
        h2_ref[:, H:] = jnp.zeros((NBLK, H), f32)

    return pl.pallas_call(
        body,
        grid=(N // NBLK,),
        in_specs=[
            pl.BlockSpec((NBLK, D), lambda i: (i, 0)),
            _full((D, H)), _full((1, H)),
            _full((D, H)), _full((1, H)),
            _full((H, H)), _full((H, H)), _full((1, H)),
        ],
        out_specs=pl.BlockSpec((NBLK, 2 * H), lambda i: (i, 0)),
        out_shape=jax.ShapeDtypeStruct((N_PAD, 2 * H), f32),
    )(x, wl, bl, wp, bp, wc1, wc2, bc)


def _edge_call(kind, xc, ea, isf, wc, we, b1, mb, wd, b2, ad, adb, sf):
    e_pad = xc.shape[0]

    def body(xc_r, ea_r, isf_r, wc_r, we_r, b1_r, mb_r,
             wd_r, b2_r, ad_r, adb_r, sf_r, out_ref):
        z = xc_r[...] @ wc_r[...] + ea_r[...] @ we_r[...] + b1_r[...]
        if kind == 'gelu':
            a = _gelu(z)
        else:
            lane = lax.broadcasted_iota(jnp.int32, (EB, 2 * H), 1)
            a = jnp.where(lane < H, _leaky(z), _gelu(z))
        mu = a @ mb_r[...]
        d = a - mu
        var = (d * d) @ mb_r[...]
        ln = d * lax.rsqrt(var + 1e-5)
        m12 = ln @ wd_r[...] + b2_r[...]
        dl = m12 @ ad_r[...] + adb_r[...]
        aw0 = jax.nn.sigmoid(dl)
        m1 = m12[:, :H]
        m2 = m12[:, H:]
        msg = m2 + aw0 * (m1 - m2)
        isfv = isf_r[...]
        scale = isfv + (1.0 - isfv) * sf_r[0, 0]
        out_ref[:, :H] = msg * scale
        out_ref[:, H:] = jnp.zeros((EB, H), f32)

    return pl.pallas_call(
        body,
        grid=(e_pad // EB,),
        in_specs=[
            pl.BlockSpec((EB, 2 * H), lambda i: (i, 0)),
            pl.BlockSpec((EB, ED), lambda i: (i, 0)),
            pl.BlockSpec((EB, 1), lambda i: (i, 0)),
            _full((2 * H, 2 * H)), _full((ED, 2 * H)),
            _full((1, 2 * H)), _full((2 * H, 2 * H)), _full((2 * H, 2 * H)),
            _full((1, 2 * H)), _full((2 * H, 1)), _full((1, 1)),
            _full((1, 1)),
        ],
        out_specs=pl.BlockSpec((EB, 2 * H), lambda i: (i, 0)),
        out_shape=jax.ShapeDtypeStruct((e_pad, 2 * H), f32),
    )(xc, ea, isf, wc, we, b1, mb, wd, b2, ad, adb, sf)


def _upd_call(kind, resid, ag, h2, acc, gwa, gwh, gb, u1a, u1h, u1b,
              u2w, u2b, attn_i):
    def body(ag_r, h_ref, acc_r, gwa_r, gwh_r, gb_r, u1a_r, u1h_r, u1b_r,
             u2w_r, u2b_r, at_r, hout_ref, accout_ref, hsum_ref):
        a = ag_r[0] + ag_r[1]
        hv = h_ref[:, :H]
        gate = jax.nn.sigmoid(a @ gwa_r[...] + hv @ gwh_r[...] + gb_r[...])
        u1 = _ln(_act(a @ u1a_r[...] + hv @ u1h_r[...] + u1b_r[...], kind))
        u2 = hv + _act(u1 @ u2w_r[...] + u2b_r[...], kind)
        hn = _ln(hv * (1.0 - gate) + u2 * gate)
        hout = hv + hn if resid else hn
        accout = acc_r[...] + at_r[0, 0] * hout
        hout_ref[:, :H] = hout
        hout_ref[:, H:] = jnp.zeros((NBLK, H), f32)
        accout_ref[...] = accout

        @pl.when(pl.program_id(0) == 0)
        def _init():
            hsum_ref[...] = jnp.zeros_like(hsum_ref)

        hsum_ref[...] += jnp.sum(accout, axis=0, keepdims=True)

    return pl.pallas_call(
        body,
        grid=(N // NBLK,),
        in_specs=[
            pl.BlockSpec((NC, NBLK, H), lambda i: (0, i, 0)),
            pl.BlockSpec((NBLK, 2 * H), lambda i: (i, 0)),
            pl.BlockSpec((NBLK, H), lambda i: (i, 0)),
            _full((H, H)), _full((H, H)), _full((1, H)),
            _full((H, H)), _full((H, H)), _full((1, H)),
            _full((H, H)), _full((1, H)), _full((1, 1)),
        ],
        out_specs=[
            pl.BlockSpec((NBLK, 2 * H), lambda i: (i, 0)),
            pl.BlockSpec((NBLK, H), lambda i: (i, 0)),
            pl.BlockSpec((1, H), lambda i: (0, 0)),
        ],
        out_shape=[
            jax.ShapeDtypeStruct((N_PAD, 2 * H), f32),
            jax.ShapeDtypeStruct((N, H), f32),
            jax.ShapeDtypeStruct((1, H), f32),
        ],
    )(ag, h2, acc, gwa, gwh, gb, u1a, u1h, u1b, u2w, u2b, attn_i)


def _head_call(hsum, w1, b1, w2, b2, w3, b3):
    def body(hs_r, w1_r, b1_r, w2_r, b2_r, w3_r, b3_r, out_ref):
        g = hs_r[...] * f32(1.0 / N)
        g = _ln(_leaky(g @ w1_r[...] + b1_r[...]))
        g = _leaky(g @ w2_r[...] + b2_r[...])
        out_ref[...] = g @ w3_r[...] + b3_r[...]

    return pl.pallas_call(
        body,
        grid=(1,),
        in_specs=[
            _full((1, H)),
            _full((H, H)), _full((1, H)),
            _full((H, H // 2)), _full((1, H // 2)),
            _full((H // 2, OUT)), _full((1, OUT)),
        ],
        out_specs=pl.BlockSpec((1, OUT), lambda i: (0, 0)),
        out_shape=jax.ShapeDtypeStruct((1, OUT), f32),
    )(hsum, w1, b1, w2, b2, w3, b3)


_MB = np.kron(np.eye(2, dtype=np.float32),
              np.full((H, H), 1.0 / H, np.float32))


def kernel(x, edge_index, edge_attr, params):
    p = params
    E = edge_index.shape[1]
    e_tot = E + N
    chunk = NW * SCB * 2
    e_pad = ((e_tot + chunk - 1) // chunk) * chunk
    padlen = e_pad - e_tot

    sl = jnp.arange(N, dtype=jnp.int32)
    src = jnp.concatenate([edge_index[0].astype(jnp.int32), sl])
    dst = jnp.concatenate([edge_index[1].astype(jnp.int32), sl])
    zpad = jnp.zeros((padlen,), jnp.int32)
    src2d = jnp.concatenate([src, zpad]).reshape(-1, SCB)
    dst2d = jnp.concatenate([dst, zpad]).reshape(-1, SCB)
    dst2d_s = jnp.concatenate(
        [dst, jnp.full((padlen,), N, jnp.int32)]).reshape(-1, SCB)

    dummy = jnp.zeros((N, ED), f32).at[:, 0].set(1.0)
    ea = jnp.concatenate(
        [edge_attr.astype(f32), dummy, jnp.zeros((padlen, ED), f32)], axis=0)

    def row(b):
        return b.reshape(1, -1).astype(f32)

    h = _emb_call(
        x.astype(f32),
        p['emb_lin_w'].T.astype(f32), row(p['emb_lin_b']),
        p['emb_pow_w'].T.astype(f32), row(p['emb_pow_b']),
        p['emb_comb_w'][:, :H].T.astype(f32),
        p['emb_comb_w'][:, H:].T.astype(f32), row(p['emb_comb_b']),
    )

    attn = jax.nn.softmax(p['layer_attn'].astype(f32))
    acc = jnp.zeros((N, H), f32)
    zeros_pad = jnp.zeros((N_PAD, H), f32)
    mb = jnp.asarray(_MB)
    zhh = jnp.zeros((H, H), f32)
    hsum = None
    isf = None

    for i in range(L):
        kind = 'gelu' if i % 2 == 1 else 'leaky'
        if i == 0:
            xc, isfw = _sc_gather(h, src2d, dst2d, True)
            isf = isfw.reshape(e_pad, 1)
        else:
            xc = _sc_gather(h, src2d, dst2d, False)
        w11 = p['mp1_w1'][i].astype(f32)
        w12 = p['mp2_w1'][i].astype(f32)
        wd = jnp.concatenate([
            jnp.concatenate([p['mp1_w2'][i].T.astype(f32), zhh], axis=1),
            jnp.concatenate([zhh, p['mp2_w2'][i].T.astype(f32)], axis=1),
        ], axis=0)
        aw_ = p['attn_w'][i].astype(f32)
        ab_ = p['attn_b'][i].astype(f32)
        wc = jnp.concatenate([
            jnp.concatenate([w11[:, :H].T, w12[:, :H].T], axis=1),
            jnp.concatenate([w11[:, H:2 * H].T, w12[:, H:2 * H].T], axis=1),
        ], axis=0)
        msg = _edge_call(
            kind, xc, ea, isf, wc,
            jnp.concatenate([w11[:, 2 * H:].T, w12[:, 2 * H:].T], axis=1),
            jnp.concatenate(
                [row(p['mp1_b1'][i]), row(p['mp2_b1'][i])], axis=1),
            mb, wd,
            jnp.concatenate(
                [row(p['mp1_b2'][i]), row(p['mp2_b2'][i])], axis=1),
            (aw_[0] - aw_[1]).reshape(2 * H, 1),
            (ab_[0] - ab_[1]).reshape(1, 1),
            p['scale_factor'][i].reshape(1, 1).astype(f32),
        )
        ag = _sc_scatter(msg, dst2d_s, zeros_pad)
        gw = p['gate_w'][i].astype(f32)
        u1w = p['upd1_w'][i].astype(f32)
        h, acc, hsum = _upd_call(
            kind, i % 2 == 1, ag, h, acc,
            gw[:, :H].T, gw[:, H:].T, row(p['gate_b'][i]),
            u1w[:, :H].T, u1w[:, H:].T, row(p['upd1_b'][i]),
            p['upd2_w'][i].T.astype(f32), row(p['upd2_b'][i]),
            attn[i].reshape(1, 1),
        )

    return _head_call(
        hsum,
        p['pre_w1'].T.astype(f32), row(p['pre_b1']),
        p['pre_w2'].T.astype(f32), row(p['pre_b2']),
        p['out_w'].T.astype(f32), row(p['out_b']),
    )


# speedup vs baseline: 1.6407x; 1.0319x over previous
import functools
import numpy as np
import jax
import jax.numpy as jnp
from jax import lax
from jax.experimental import pallas as pl
from jax.experimental.pallas import tpu as pltpu
from jax.experimental.pallas import tpu_sc as plsc

N = 10000
D = 128
H = 64
ED = 4
L = 4
OUT = 4

NC, NS = 2, 16
NW = NC * NS
SCB = 128
EB = 8192
NBLK = 2000
N_PAD = 10240
RPS = N_PAD // NS

f32 = jnp.float32
bf16 = jnp.bfloat16


def _ln(x):
    m = jnp.mean(x, axis=-1, keepdims=True)
    v = jnp.mean((x - m) ** 2, axis=-1, keepdims=True)
    return (x - m) / jnp.sqrt(v + 1e-5)


def _leaky(x):
    return jnp.where(x >= 0, x, 0.1 * x)


def _gelu(x):
    return 0.5 * x * (1.0 + lax.erf(x / jnp.sqrt(jnp.float32(2.0))))


def _act(x, kind):
    return _gelu(x) if kind == 'gelu' else _leaky(x)


def _sc_mesh():
    return plsc.VectorSubcoreMesh(
        core_axis_name="c", subcore_axis_name="s",
        num_cores=NC, num_subcores=NS)


def _sc_gather(h2, src2d, dst2d, with_isf):
    nb = src2d.shape[0] // NW
    e_pad = NW * nb * SCB
    npairs = nb // 2

    def body(h_hbm, src_hbm, dst_hbm, *refs):
        if with_isf:
            (xc_hbm, isf_hbm, idx_d, idx_s, ri_a, rj_a, ri_b, rj_b,
             isf_v, hs, sem_ga, sem_gb, sem_wa, sem_wb) = refs
        else:
            (xc_hbm, idx_d, idx_s, ri_a, rj_a, ri_b, rj_b,
             hs, sem_ga, sem_gb, sem_wa, sem_wb) = refs
        c = lax.axis_index("c")
        s = lax.axis_index("s")
        wid = s * NC + c
        pltpu.sync_copy(h_hbm.at[pl.ds(s * RPS, RPS), pl.ds(0, H)],
                        hs.at[pl.ds(s * RPS, RPS)])
        pltpu.sync_copy(dst_hbm.at[pl.ds(wid * nb, nb)], idx_d)
        pltpu.sync_copy(src_hbm.at[pl.ds(wid * nb, nb)], idx_s)
        plsc.subcore_barrier()

        if with_isf:
            def isf_step(j, carry):
                for k in range(SCB // 16):
                    d16 = idx_d[j, pl.ds(k * 16, 16)]
                    s16 = idx_s[j, pl.ds(k * 16, 16)]
                    isf_v[j, pl.ds(k * 16, 16)] = jnp.where(
                        d16 == s16, f32(1.0), f32(0.0))
                return carry
            lax.fori_loop(0, nb, isf_step, 0)
            pltpu.sync_copy(
                isf_v, isf_hbm.at[pl.ds(wid * nb, nb)])

        base0 = wid * nb * SCB

        def gather(j, ri, rj, sem):
            pltpu.async_copy(hs.at[idx_d.at[j]], ri, sem)
            pltpu.async_copy(hs.at[idx_s.at[j]], rj, sem)

        def drain2(sem):
            pltpu.make_async_copy(
                xc_hbm.at[pl.ds(0, SCB), pl.ds(0, H)], ri_a, sem).wait()
            pltpu.make_async_copy(
                xc_hbm.at[pl.ds(0, SCB), pl.ds(0, H)], ri_a, sem).wait()

        def write(j, ri, rj, sem):
            base = base0 + j * SCB
            pltpu.async_copy(ri, xc_hbm.at[pl.ds(base, SCB), pl.ds(0, H)], sem)
            pltpu.async_copy(rj, xc_hbm.at[pl.ds(base, SCB), pl.ds(H, H)], sem)

        gather(0, ri_a, rj_a, sem_ga)

        def step(jj, carry):
            j0 = 2 * jj
            j1 = 2 * jj + 1

            @pl.when(jj > 0)
            def _():
                drain2(sem_wb)
            gather(j1, ri_b, rj_b, sem_gb)
            drain2(sem_ga)
            write(j0, ri_a, rj_a, sem_wa)
            drain2(sem_wa)

            @pl.when(jj + 1 < npairs)
            def _():
                gather(j0 + 2, ri_a, rj_a, sem_ga)
            drain2(sem_gb)
            write(j1, ri_b, rj_b, sem_wb)
            return carry

        lax.fori_loop(0, npairs, step, 0)
        drain2(sem_wb)

    out_type = [
        jax.ShapeDtypeStruct((e_pad, 2 * H), f32),
    ]
    scratch = [
        pltpu.VMEM((nb, SCB), jnp.int32),
        pltpu.VMEM((nb, SCB), jnp.int32),
        pltpu.VMEM((SCB, H), f32),
        pltpu.VMEM((SCB, H), f32),
        pltpu.VMEM((SCB, H), f32),
        pltpu.VMEM((SCB, H), f32),
    ]
    if with_isf:
        out_type.append(jax.ShapeDtypeStruct((NW * nb, SCB), f32))
        scratch.append(pltpu.VMEM((nb, SCB), f32))
    scratch.append(pltpu.VMEM_SHARED((N_PAD, H), f32))
    scratch += [pltpu.SemaphoreType.DMA] * 4

    return pl.kernel(
        body,
        out_type=tuple(out_type) if with_isf else out_type[0],
        mesh=_sc_mesh(),
        scratch_types=scratch,
        compiler_params=pltpu.CompilerParams(use_tc_tiling_on_sc=False),
    )(h2, src2d, dst2d)


def _sc_scatter(msg, dst2d, zeros_pad):
    nb = dst2d.shape[0] // NW
    npairs = nb // 2

    def body(msg_hbm, dst_hbm, z_hbm, out_hbm, idx_v, msg_a, msg_b, acc,
             sem_ma, sem_mb, sem_sa, sem_sb):
        c = lax.axis_index("c")
        s = lax.axis_index("s")
        wid = s * NC + c
        pltpu.sync_copy(z_hbm.at[pl.ds(s * RPS, RPS)],
                        acc.at[pl.ds(s * RPS, RPS)])
        pltpu.sync_copy(dst_hbm.at[pl.ds(wid * nb, nb)], idx_v)
        plsc.subcore_barrier()
        base0 = wid * nb * SCB

        def drain1(buf, sem):
            pltpu.make_async_copy(
                msg_hbm.at[pl.ds(0, SCB), pl.ds(0, H)], buf, sem).wait()

        pltpu.async_copy(
            msg_hbm.at[pl.ds(base0, SCB), pl.ds(0, H)], msg_a, sem_ma)

        def step(jj, carry):
            j0 = 2 * jj
            j1 = 2 * jj + 1

            @pl.when(jj > 0)
            def _():
                drain1(msg_b, sem_sb)
            pltpu.async_copy(
                msg_hbm.at[pl.ds(base0 + j1 * SCB, SCB), pl.ds(0, H)],
                msg_b, sem_mb)
            drain1(msg_a, sem_ma)
            pltpu.async_copy(msg_a, acc.at[idx_v.at[j0]], sem_sa, add=True)
            drain1(msg_a, sem_sa)

            @pl.when(jj + 1 < npairs)
            def _():
                pltpu.async_copy(
                    msg_hbm.at[pl.ds(base0 + (j0 + 2) * SCB, SCB),
                               pl.ds(0, H)],
                    msg_a, sem_ma)
            drain1(msg_b, sem_mb)
            pltpu.async_copy(msg_b, acc.at[idx_v.at[j1]], sem_sb, add=True)
            return carry

        lax.fori_loop(0, npairs, step, 0)
        drain1(msg_b, sem_sb)
        plsc.subcore_barrier()
        pltpu.sync_copy(acc.at[pl.ds(s * RPS, RPS)],
                        out_hbm.at[c, pl.ds(s * RPS, RPS)])

    return pl.kernel(
        body,
        out_type=jax.ShapeDtypeStruct((NC, N_PAD, H), f32),
        mesh=_sc_mesh(),
        scratch_types=[
            pltpu.VMEM((nb, SCB), jnp.int32),
            pltpu.VMEM((SCB, H), f32),
            pltpu.VMEM((SCB, H), f32),
            pltpu.VMEM_SHARED((N_PAD, H), f32),
            pltpu.SemaphoreType.DMA,
            pltpu.SemaphoreType.DMA,
            pltpu.SemaphoreType.DMA,
            pltpu.SemaphoreType.DMA,
        ],
        compiler_params=pltpu.CompilerParams(use_tc_tiling_on_sc=False),
    )(msg, dst2d, zeros_pad)


def _full(shape):
    return pl.BlockSpec(shape, lambda i: (0,) * len(shape))


def _emb_call(x, wl, bl, wp, bp, wc1, wc2, bc):
    def body(x_ref, wl_r, bl_r, wp_r, bp_r, wc1_r, wc2_r, bc_r, h2_ref):
        xv = x_ref[...]
        lin = xv @ wl_r[...] + bl_r[...]
        pw = (xv * xv) @ wp_r[...] + bp_r[...]
        h2_ref[:, :H] = lin @ wc1_r[...] + pw @ wc2_r[...] + bc_r[...]
        h2_ref[:, H:] = jnp.zeros((NBLK, H), f32)

    return pl.pallas_call(
        body,
        grid=(N // NBLK,),
        in_specs=[
            pl.BlockSpec((NBLK, D), lambda i: (i, 0)),
            _full((D, H)), _full((1, H)),
            _full((D, H)), _full((1, H)),
            _full((H, H)), _full((H, H)), _full((1, H)),
        ],
        out_specs=pl.BlockSpec((NBLK, 2 * H), lambda i: (i, 0)),
        out_shape=jax.ShapeDtypeStruct((N_PAD, 2 * H), f32),
    )(x, wl, bl, wp, bp, wc1, wc2, bc)


def _edge_call(kind, xc, ea, isf, wc, we, b1, mb, wd, b2, ad, adb, sf):
    e_pad = xc.shape[0]

    def body(xc_r, ea_r, isf_r, wc_r, we_r, b1_r, mb_r,
             wd_r, b2_r, ad_r, adb_r, sf_r, out_ref):
        z = xc_r[...] @ wc_r[...] + ea_r[...] @ we_r[...] + b1_r[...]
        if kind == 'gelu':
            a = _gelu(z)
        else:
            lane = lax.broadcasted_iota(jnp.int32, (EB, 2 * H), 1)
            a = jnp.where(lane < H, _leaky(z), _gelu(z))
        mu = a @ mb_r[...]
        d = a - mu
        var = (d * d) @ mb_r[...]
        ln = d * lax.rsqrt(var + 1e-5)
        m12 = ln @ wd_r[...] + b2_r[...]
        dl = m12 @ ad_r[...] + adb_r[...]
        aw0 = jax.nn.sigmoid(dl)
        m1 = m12[:, :H]
        m2 = m12[:, H:]
        msg = m2 + aw0 * (m1 - m2)
        isfv = isf_r[...]
        scale = isfv + (1.0 - isfv) * sf_r[0, 0]
        out_ref[:, :H] = msg * scale

    return pl.pallas_call(
        body,
        grid=(e_pad // EB,),
        in_specs=[
            pl.BlockSpec((EB, 2 * H), lambda i: (i, 0)),
            pl.BlockSpec((EB, ED), lambda i: (i, 0)),
            pl.BlockSpec((EB, 1), lambda i: (i, 0)),
            _full((2 * H, 2 * H)), _full((ED, 2 * H)),
            _full((1, 2 * H)), _full((2 * H, 2 * H)), _full((2 * H, 2 * H)),
            _full((1, 2 * H)), _full((2 * H, 1)), _full((1, 1)),
            _full((1, 1)),
        ],
        out_specs=pl.BlockSpec((EB, 2 * H), lambda i: (i, 0)),
        out_shape=jax.ShapeDtypeStruct((e_pad, 2 * H), f32),
    )(xc, ea, isf, wc, we, b1, mb, wd, b2, ad, adb, sf)


def _upd_call(kind, resid, ag, h2, acc, gwa, gwh, gb, u1a, u1h, u1b,
              u2w, u2b, attn_i):
    def body(ag_r, h_ref, acc_r, gwa_r, gwh_r, gb_r, u1a_r, u1h_r, u1b_r,
             u2w_r, u2b_r, at_r, hout_ref, accout_ref, hsum_ref):
        a = ag_r[0] + ag_r[1]
        hv = h_ref[:, :H]
        gate = jax.nn.sigmoid(a @ gwa_r[...] + hv @ gwh_r[...] + gb_r[...])
        u1 = _ln(_act(a @ u1a_r[...] + hv @ u1h_r[...] + u1b_r[...], kind))
        u2 = hv + _act(u1 @ u2w_r[...] + u2b_r[...], kind)
        hn = _ln(hv * (1.0 - gate) + u2 * gate)
        hout = hv + hn if resid else hn
        accout = acc_r[...] + at_r[0, 0] * hout
        hout_ref[:, :H] = hout
        hout_ref[:, H:] = jnp.zeros((NBLK, H), f32)
        accout_ref[...] = accout

        @pl.when(pl.program_id(0) == 0)
        def _init():
            hsum_ref[...] = jnp.zeros_like(hsum_ref)

        hsum_ref[...] += jnp.sum(accout, axis=0, keepdims=True)

    return pl.pallas_call(
        body,
        grid=(N // NBLK,),
        in_specs=[
            pl.BlockSpec((NC, NBLK, H), lambda i: (0, i, 0)),
            pl.BlockSpec((NBLK, 2 * H), lambda i: (i, 0)),
            pl.BlockSpec((NBLK, H), lambda i: (i, 0)),
            _full((H, H)), _full((H, H)), _full((1, H)),
            _full((H, H)), _full((H, H)), _full((1, H)),
            _full((H, H)), _full((1, H)), _full((1, 1)),
        ],
        out_specs=[
            pl.BlockSpec((NBLK, 2 * H), lambda i: (i, 0)),
            pl.BlockSpec((NBLK, H), lambda i: (i, 0)),
            pl.BlockSpec((1, H), lambda i: (0, 0)),
        ],
        out_shape=[
            jax.ShapeDtypeStruct((N_PAD, 2 * H), f32),
            jax.ShapeDtypeStruct((N, H), f32),
            jax.ShapeDtypeStruct((1, H), f32),
        ],
    )(ag, h2, acc, gwa, gwh, gb, u1a, u1h, u1b, u2w, u2b, attn_i)


def _head_call(hsum, w1, b1, w2, b2, w3, b3):
    def body(hs_r, w1_r, b1_r, w2_r, b2_r, w3_r, b3_r, out_ref):
        g = hs_r[...] * f32(1.0 / N)
        g = _ln(_leaky(g @ w1_r[...] + b1_r[...]))
        g = _leaky(g @ w2_r[...] + b2_r[...])
        out_ref[...] = g @ w3_r[...] + b3_r[...]

    return pl.pallas_call(
        body,
        grid=(1,),
        in_specs=[
            _full((1, H)),
            _full((H, H)), _full((1, H)),
            _full((H, H // 2)), _full((1, H // 2)),
            _full((H // 2, OUT)), _full((1, OUT)),
        ],
        out_specs=pl.BlockSpec((1, OUT), lambda i: (0, 0)),
        out_shape=jax.ShapeDtypeStruct((1, OUT), f32),
    )(hsum, w1, b1, w2, b2, w3, b3)


_MB = np.kron(np.eye(2, dtype=np.float32),
              np.full((H, H), 1.0 / H, np.float32))


def kernel(x, edge_index, edge_attr, params):
    p = params
    E = edge_index.shape[1]
    e_tot = E + N
    chunk = NW * SCB * 2
    e_pad = ((e_tot + chunk - 1) // chunk) * chunk
    padlen = e_pad - e_tot

    sl = jnp.arange(N, dtype=jnp.int32)
    src = jnp.concatenate([edge_index[0].astype(jnp.int32), sl])
    dst = jnp.concatenate([edge_index[1].astype(jnp.int32), sl])
    zpad = jnp.zeros((padlen,), jnp.int32)
    src2d = jnp.concatenate([src, zpad]).reshape(-1, SCB)
    dst2d = jnp.concatenate([dst, zpad]).reshape(-1, SCB)
    dst2d_s = jnp.concatenate(
        [dst, jnp.full((padlen,), N, jnp.int32)]).reshape(-1, SCB)

    dummy = jnp.zeros((N, ED), f32).at[:, 0].set(1.0)
    ea = jnp.concatenate(
        [edge_attr.astype(f32), dummy, jnp.zeros((padlen, ED), f32)], axis=0)

    def row(b):
        return b.reshape(1, -1).astype(f32)

    h = _emb_call(
        x.astype(f32),
        p['emb_lin_w'].T.astype(f32), row(p['emb_lin_b']),
        p['emb_pow_w'].T.astype(f32), row(p['emb_pow_b']),
        p['emb_comb_w'][:, :H].T.astype(f32),
        p['emb_comb_w'][:, H:].T.astype(f32), row(p['emb_comb_b']),
    )

    attn = jax.nn.softmax(p['layer_attn'].astype(f32))
    acc = jnp.zeros((N, H), f32)
    zeros_pad = jnp.zeros((N_PAD, H), f32)
    mb = jnp.asarray(_MB)
    zhh = jnp.zeros((H, H), f32)
    hsum = None
    isf = None

    for i in range(L):
        kind = 'gelu' if i % 2 == 1 else 'leaky'
        if i == 0:
            xc, isfw = _sc_gather(h, src2d, dst2d, True)
            isf = isfw.reshape(e_pad, 1)
        else:
            xc = _sc_gather(h, src2d, dst2d, False)
        w11 = p['mp1_w1'][i].astype(f32)
        w12 = p['mp2_w1'][i].astype(f32)
        wd = jnp.concatenate([
            jnp.concatenate([p['mp1_w2'][i].T.astype(f32), zhh], axis=1),
            jnp.concatenate([zhh, p['mp2_w2'][i].T.astype(f32)], axis=1),
        ], axis=0)
        aw_ = p['attn_w'][i].astype(f32)
        ab_ = p['attn_b'][i].astype(f32)
        wc = jnp.concatenate([
            jnp.concatenate([w11[:, :H].T, w12[:, :H].T], axis=1),
            jnp.concatenate([w11[:, H:2 * H].T, w12[:, H:2 * H].T], axis=1),
        ], axis=0)
        msg = _edge_call(
            kind, xc, ea, isf, wc,
            jnp.concatenate([w11[:, 2 * H:].T, w12[:, 2 * H:].T], axis=1),
            jnp.concatenate(
                [row(p['mp1_b1'][i]), row(p['mp2_b1'][i])], axis=1),
            mb, wd,
            jnp.concatenate(
                [row(p['mp1_b2'][i]), row(p['mp2_b2'][i])], axis=1),
            (aw_[0] - aw_[1]).reshape(2 * H, 1),
            (ab_[0] - ab_[1]).reshape(1, 1),
            p['scale_factor'][i].reshape(1, 1).astype(f32),
        )
        ag = _sc_scatter(msg, dst2d_s, zeros_pad)
        gw = p['gate_w'][i].astype(f32)
        u1w = p['upd1_w'][i].astype(f32)
        h, acc, hsum = _upd_call(
            kind, i % 2 == 1, ag, h, acc,
            gw[:, :H].T, gw[:, H:].T, row(p['gate_b'][i]),
            u1w[:, :H].T, u1w[:, H:].T, row(p['upd1_b'][i]),
            p['upd2_w'][i].T.astype(f32), row(p['upd2_b'][i]),
            attn[i].reshape(1, 1),
        )

    return _head_call(
        hsum,
        p['pre_w1'].T.astype(f32), row(p['pre_b1']),
        p['pre_w2'].T.astype(f32), row(p['pre_b2']),
        p['out_w'].T.astype(f32), row(p['out_b']),
    )


# speedup vs baseline: 1.6411x; 1.0003x over previous
import numpy as np
import jax
import jax.numpy as jnp
from jax import lax
from jax.experimental import pallas as pl
from jax.experimental.pallas import tpu as pltpu
from jax.experimental.pallas import tpu_sc as plsc

N = 10000
D = 128
H = 64
ED = 4
L = 4
OUT = 4

NC, NS = 2, 16
NW = NC * NS
SCB = 128
EB = 8192
NBLK = 2000
N_PAD = 10240
RPS = N_PAD // NS

f32 = jnp.float32


def _ln(x):
    m = jnp.mean(x, axis=-1, keepdims=True)
    v = jnp.mean((x - m) ** 2, axis=-1, keepdims=True)
    return (x - m) / jnp.sqrt(v + 1e-5)


def _leaky(x):
    return jnp.where(x >= 0, x, 0.1 * x)


def _gelu(x):
    return 0.5 * x * (1.0 + lax.erf(x / jnp.sqrt(jnp.float32(2.0))))


def _act(x, kind):
    return _gelu(x) if kind == 'gelu' else _leaky(x)


def _sc_mesh():
    return plsc.VectorSubcoreMesh(
        core_axis_name="c", subcore_axis_name="s",
        num_cores=NC, num_subcores=NS)


def _sc_gather(h2, src2d, dst2d, with_isf):
    nb = src2d.shape[0] // NW
    e_pad = NW * nb * SCB
    npairs = nb // 2

    def body(h_hbm, src_hbm, dst_hbm, *refs):
        if with_isf:
            (xc_hbm, isf_hbm, idx_d, idx_s, ri_a, rj_a, ri_b, rj_b,
             isf_v, hs, sem_ga, sem_gb, sem_wa, sem_wb) = refs
        else:
            (xc_hbm, idx_d, idx_s, ri_a, rj_a, ri_b, rj_b,
             hs, sem_ga, sem_gb, sem_wa, sem_wb) = refs
        c = lax.axis_index("c")
        s = lax.axis_index("s")
        wid = s * NC + c
        pltpu.sync_copy(h_hbm.at[pl.ds(s * RPS, RPS), pl.ds(0, H)],
                        hs.at[pl.ds(s * RPS, RPS)])
        pltpu.sync_copy(dst_hbm.at[pl.ds(wid * nb, nb)], idx_d)
        pltpu.sync_copy(src_hbm.at[pl.ds(wid * nb, nb)], idx_s)
        plsc.subcore_barrier()

        if with_isf:
            def isf_step(j, carry):
                for k in range(SCB // 16):
                    d16 = idx_d[j, pl.ds(k * 16, 16)]
                    s16 = idx_s[j, pl.ds(k * 16, 16)]
                    isf_v[j, pl.ds(k * 16, 16)] = jnp.where(
                        d16 == s16, f32(1.0), f32(0.0))
                return carry
            lax.fori_loop(0, nb, isf_step, 0)
            pltpu.sync_copy(
                isf_v, isf_hbm.at[pl.ds(wid * nb, nb)])

        base0 = wid * nb * SCB

        def gather(j, ri, rj, sem):
            pltpu.async_copy(hs.at[idx_d.at[j]], ri, sem)
            pltpu.async_copy(hs.at[idx_s.at[j]], rj, sem)

        def drain2(sem):
            pltpu.make_async_copy(
                xc_hbm.at[pl.ds(0, SCB), pl.ds(0, H)], ri_a, sem).wait()
            pltpu.make_async_copy(
                xc_hbm.at[pl.ds(0, SCB), pl.ds(0, H)], ri_a, sem).wait()

        def write(j, ri, rj, sem):
            base = base0 + j * SCB
            pltpu.async_copy(ri, xc_hbm.at[pl.ds(base, SCB), pl.ds(0, H)], sem)
            pltpu.async_copy(rj, xc_hbm.at[pl.ds(base, SCB), pl.ds(H, H)], sem)

        gather(0, ri_a, rj_a, sem_ga)

        def step(jj, carry):
            j0 = 2 * jj
            j1 = 2 * jj + 1

            @pl.when(jj > 0)
            def _():
                drain2(sem_wb)
            gather(j1, ri_b, rj_b, sem_gb)
            drain2(sem_ga)
            write(j0, ri_a, rj_a, sem_wa)
            drain2(sem_wa)

            @pl.when(jj + 1 < npairs)
            def _():
                gather(j0 + 2, ri_a, rj_a, sem_ga)
            drain2(sem_gb)
            write(j1, ri_b, rj_b, sem_wb)
            return carry

        lax.fori_loop(0, npairs, step, 0)
        drain2(sem_wb)

    out_type = [
        jax.ShapeDtypeStruct((e_pad, 2 * H), f32),
    ]
    scratch = [
        pltpu.VMEM((nb, SCB), jnp.int32),
        pltpu.VMEM((nb, SCB), jnp.int32),
        pltpu.VMEM((SCB, H), f32),
        pltpu.VMEM((SCB, H), f32),
        pltpu.VMEM((SCB, H), f32),
        pltpu.VMEM((SCB, H), f32),
    ]
    if with_isf:
        out_type.append(jax.ShapeDtypeStruct((NW * nb, SCB), f32))
        scratch.append(pltpu.VMEM((nb, SCB), f32))
    scratch.append(pltpu.VMEM_SHARED((N_PAD, H), f32))
    scratch += [pltpu.SemaphoreType.DMA] * 4

    return pl.kernel(
        body,
        out_type=tuple(out_type) if with_isf else out_type[0],
        mesh=_sc_mesh(),
        scratch_types=scratch,
        compiler_params=pltpu.CompilerParams(use_tc_tiling_on_sc=False),
    )(h2, src2d, dst2d)


def _sc_scatter(msg, dst2d, zeros_pad):
    nb = dst2d.shape[0] // NW
    npairs = nb // 2

    def body(msg_hbm, dst_hbm, z_hbm, out_hbm, idx_v, msg_a, msg_b, acc,
             sem_ma, sem_mb, sem_sa, sem_sb):
        c = lax.axis_index("c")
        s = lax.axis_index("s")
        wid = s * NC + c
        pltpu.sync_copy(z_hbm.at[pl.ds(s * RPS, RPS)],
                        acc.at[pl.ds(s * RPS, RPS)])
        pltpu.sync_copy(dst_hbm.at[pl.ds(wid * nb, nb)], idx_v)
        plsc.subcore_barrier()
        base0 = wid * nb * SCB

        def drain1(buf, sem):
            pltpu.make_async_copy(
                msg_hbm.at[pl.ds(0, SCB), pl.ds(0, H)], buf, sem).wait()

        pltpu.async_copy(
            msg_hbm.at[pl.ds(base0, SCB), pl.ds(0, H)], msg_a, sem_ma)

        def step(jj, carry):
            j0 = 2 * jj
            j1 = 2 * jj + 1

            @pl.when(jj > 0)
            def _():
                drain1(msg_b, sem_sb)
            pltpu.async_copy(
                msg_hbm.at[pl.ds(base0 + j1 * SCB, SCB), pl.ds(0, H)],
                msg_b, sem_mb)
            drain1(msg_a, sem_ma)
            pltpu.async_copy(msg_a, acc.at[idx_v.at[j0]], sem_sa, add=True)
            drain1(msg_a, sem_sa)

            @pl.when(jj + 1 < npairs)
            def _():
                pltpu.async_copy(
                    msg_hbm.at[pl.ds(base0 + (j0 + 2) * SCB, SCB),
                               pl.ds(0, H)],
                    msg_a, sem_ma)
            drain1(msg_b, sem_mb)
            pltpu.async_copy(msg_b, acc.at[idx_v.at[j1]], sem_sb, add=True)
            return carry

        lax.fori_loop(0, npairs, step, 0)
        drain1(msg_b, sem_sb)
        plsc.subcore_barrier()
        pltpu.sync_copy(acc.at[pl.ds(s * RPS, RPS)],
                        out_hbm.at[c, pl.ds(s * RPS, RPS)])

    return pl.kernel(
        body,
        out_type=jax.ShapeDtypeStruct((NC, N_PAD, H), f32),
        mesh=_sc_mesh(),
        scratch_types=[
            pltpu.VMEM((nb, SCB), jnp.int32),
            pltpu.VMEM((SCB, H), f32),
            pltpu.VMEM((SCB, H), f32),
            pltpu.VMEM_SHARED((N_PAD, H), f32),
            pltpu.SemaphoreType.DMA,
            pltpu.SemaphoreType.DMA,
            pltpu.SemaphoreType.DMA,
            pltpu.SemaphoreType.DMA,
        ],
        compiler_params=pltpu.CompilerParams(use_tc_tiling_on_sc=False),
    )(msg, dst2d, zeros_pad)


def _full(shape):
    return pl.BlockSpec(shape, lambda i: (0,) * len(shape))


def _emb_call(x, wl, bl, wp, bp, wc1, wc2, bc):
    def body(x_ref, wl_r, bl_r, wp_r, bp_r, wc1_r, wc2_r, bc_r, h2_ref):
        xv = x_ref[...]
        lin = xv @ wl_r[...] + bl_r[...]
        pw = (xv * xv) @ wp_r[...] + bp_r[...]
        h2_ref[:, :H] = lin @ wc1_r[...] + pw @ wc2_r[...] + bc_r[...]
        h2_ref[:, H:] = jnp.zeros((NBLK, H), f32)

    return pl.pallas_call(
        body,
        grid=(N // NBLK,),
        in_specs=[
            pl.BlockSpec((NBLK, D), lambda i: (i, 0)),
            _full((D, H)), _full((1, H)),
            _full((D, H)), _full((1, H)),
            _full((H, H)), _full((H, H)), _full((1, H)),
        ],
        out_specs=pl.BlockSpec((NBLK, 2 * H), lambda i: (i, 0)),
        out_shape=jax.ShapeDtypeStruct((N_PAD, 2 * H), f32),
    )(x, wl, bl, wp, bp, wc1, wc2, bc)


def _edge_call(kind, xc, ea, isf, wc, we, b1, mb, wd, b2, ad, adb, sf):
    e_pad = xc.shape[0]

    def body(xc_r, ea_r, isf_r, wc_r, we_r, b1_r, mb_r,
             wd_r, b2_r, ad_r, adb_r, sf_r, out_ref):
        z = xc_r[...] @ wc_r[...] + ea_r[...] @ we_r[...] + b1_r[...]
        if kind == 'gelu':
            a = _gelu(z)
        else:
            lane = lax.broadcasted_iota(jnp.int32, (EB, 2 * H), 1)
            a = jnp.where(lane < H, _leaky(z), _gelu(z))
        mu = a @ mb_r[...]
        d = a - mu
        var = (d * d) @ mb_r[...]
        ln = d * lax.rsqrt(var + 1e-5)
        m12 = ln @ wd_r[...] + b2_r[...]
        dl = m12 @ ad_r[...] + adb_r[...]
        aw0 = jax.nn.sigmoid(dl)
        m1 = m12[:, :H]
        m2 = m12[:, H:]
        msg = m2 + aw0 * (m1 - m2)
        isfv = isf_r[...]
        scale = isfv + (1.0 - isfv) * sf_r[0, 0]
        out_ref[:, :H] = msg * scale

    return pl.pallas_call(
        body,
        grid=(e_pad // EB,),
        in_specs=[
            pl.BlockSpec((EB, 2 * H), lambda i: (i, 0)),
            pl.BlockSpec((EB, ED), lambda i: (i, 0)),
            pl.BlockSpec((EB, 1), lambda i: (i, 0)),
            _full((2 * H, 2 * H)), _full((ED, 2 * H)),
            _full((1, 2 * H)), _full((2 * H, 2 * H)), _full((2 * H, 2 * H)),
            _full((1, 2 * H)), _full((2 * H, 1)), _full((1, 1)),
            _full((1, 1)),
        ],
        out_specs=pl.BlockSpec((EB, 2 * H), lambda i: (i, 0)),
        out_shape=jax.ShapeDtypeStruct((e_pad, 2 * H), f32),
    )(xc, ea, isf, wc, we, b1, mb, wd, b2, ad, adb, sf)


def _upd_call(kind, resid, ag, h2, acc, gwa, gwh, gb, u1a, u1h, u1b,
              u2w, u2b, attn_i):
    def body(ag_r, h_ref, acc_r, gwa_r, gwh_r, gb_r, u1a_r, u1h_r, u1b_r,
             u2w_r, u2b_r, at_r, hout_ref, accout_ref, hsum_ref):
        a = ag_r[0] + ag_r[1]
        hv = h_ref[:, :H]
        gate = jax.nn.sigmoid(a @ gwa_r[...] + hv @ gwh_r[...] + gb_r[...])
        u1 = _ln(_act(a @ u1a_r[...] + hv @ u1h_r[...] + u1b_r[...], kind))
        u2 = hv + _act(u1 @ u2w_r[...] + u2b_r[...], kind)
        hn = _ln(hv * (1.0 - gate) + u2 * gate)
        hout = hv + hn if resid else hn
        accout = acc_r[...] + at_r[0, 0] * hout
        hout_ref[:, :H] = hout
        hout_ref[:, H:] = jnp.zeros((NBLK, H), f32)
        accout_ref[...] = accout

        @pl.when(pl.program_id(0) == 0)
        def _init():
            hsum_ref[...] = jnp.zeros_like(hsum_ref)

        hsum_ref[...] += jnp.sum(accout, axis=0, keepdims=True)

    return pl.pallas_call(
        body,
        grid=(N // NBLK,),
        in_specs=[
            pl.BlockSpec((NC, NBLK, H), lambda i: (0, i, 0)),
            pl.BlockSpec((NBLK, 2 * H), lambda i: (i, 0)),
            pl.BlockSpec((NBLK, H), lambda i: (i, 0)),
            _full((H, H)), _full((H, H)), _full((1, H)),
            _full((H, H)), _full((H, H)), _full((1, H)),
            _full((H, H)), _full((1, H)), _full((1, 1)),
        ],
        out_specs=[
            pl.BlockSpec((NBLK, 2 * H), lambda i: (i, 0)),
            pl.BlockSpec((NBLK, H), lambda i: (i, 0)),
            pl.BlockSpec((1, H), lambda i: (0, 0)),
        ],
        out_shape=[
            jax.ShapeDtypeStruct((N_PAD, 2 * H), f32),
            jax.ShapeDtypeStruct((N, H), f32),
            jax.ShapeDtypeStruct((1, H), f32),
        ],
    )(ag, h2, acc, gwa, gwh, gb, u1a, u1h, u1b, u2w, u2b, attn_i)


def _head_call(hsum, w1, b1, w2, b2, w3, b3):
    def body(hs_r, w1_r, b1_r, w2_r, b2_r, w3_r, b3_r, out_ref):
        g = hs_r[...] * f32(1.0 / N)
        g = _ln(_leaky(g @ w1_r[...] + b1_r[...]))
        g = _leaky(g @ w2_r[...] + b2_r[...])
        out_ref[...] = g @ w3_r[...] + b3_r[...]

    return pl.pallas_call(
        body,
        grid=(1,),
        in_specs=[
            _full((1, H)),
            _full((H, H)), _full((1, H)),
            _full((H, H // 2)), _full((1, H // 2)),
            _full((H // 2, OUT)), _full((1, OUT)),
        ],
        out_specs=pl.BlockSpec((1, OUT), lambda i: (0, 0)),
        out_shape=jax.ShapeDtypeStruct((1, OUT), f32),
    )(hsum, w1, b1, w2, b2, w3, b3)


_MB = np.kron(np.eye(2, dtype=np.float32),
              np.full((H, H), 1.0 / H, np.float32))


def kernel(x, edge_index, edge_attr, params):
    p = params
    E = edge_index.shape[1]
    e_tot = E + N
    chunk = NW * SCB * 2
    e_pad = ((e_tot + chunk - 1) // chunk) * chunk
    padlen = e_pad - e_tot

    sl = jnp.arange(N, dtype=jnp.int32)
    src = jnp.concatenate([edge_index[0].astype(jnp.int32), sl])
    dst = jnp.concatenate([edge_index[1].astype(jnp.int32), sl])
    zpad = jnp.zeros((padlen,), jnp.int32)
    src2d = jnp.concatenate([src, zpad]).reshape(-1, SCB)
    dst2d = jnp.concatenate([dst, zpad]).reshape(-1, SCB)
    dst2d_s = jnp.concatenate(
        [dst, jnp.full((padlen,), N, jnp.int32)]).reshape(-1, SCB)

    dummy = jnp.zeros((N, ED), f32).at[:, 0].set(1.0)
    ea = jnp.concatenate(
        [edge_attr.astype(f32), dummy, jnp.zeros((padlen, ED), f32)], axis=0)

    def row(b):
        return b.reshape(1, -1).astype(f32)

    h = _emb_call(
        x.astype(f32),
        p['emb_lin_w'].T.astype(f32), row(p['emb_lin_b']),
        p['emb_pow_w'].T.astype(f32), row(p['emb_pow_b']),
        p['emb_comb_w'][:, :H].T.astype(f32),
        p['emb_comb_w'][:, H:].T.astype(f32), row(p['emb_comb_b']),
    )

    attn = jax.nn.softmax(p['layer_attn'].astype(f32))
    acc = jnp.zeros((N, H), f32)
    zeros_pad = jnp.zeros((N_PAD, H), f32)
    mb = jnp.asarray(_MB)
    zhh = jnp.zeros((H, H), f32)
    hsum = None
    isf = None

    for i in range(L):
        kind = 'gelu' if i % 2 == 1 else 'leaky'
        if i == 0:
            xc, isfw = _sc_gather(h, src2d, dst2d, True)
            isf = isfw.reshape(e_pad, 1)
        else:
            xc = _sc_gather(h, src2d, dst2d, False)
        w11 = p['mp1_w1'][i].astype(f32)
        w12 = p['mp2_w1'][i].astype(f32)
        wd = jnp.concatenate([
            jnp.concatenate([p['mp1_w2'][i].T.astype(f32), zhh], axis=1),
            jnp.concatenate([zhh, p['mp2_w2'][i].T.astype(f32)], axis=1),
        ], axis=0)
        aw_ = p['attn_w'][i].astype(f32)
        ab_ = p['attn_b'][i].astype(f32)
        wc = jnp.concatenate([
            jnp.concatenate([w11[:, :H].T, w12[:, :H].T], axis=1),
            jnp.concatenate([w11[:, H:2 * H].T, w12[:, H:2 * H].T], axis=1),
        ], axis=0)
        msg = _edge_call(
            kind, xc, ea, isf, wc,
            jnp.concatenate([w11[:, 2 * H:].T, w12[:, 2 * H:].T], axis=1),
            jnp.concatenate(
                [row(p['mp1_b1'][i]), row(p['mp2_b1'][i])], axis=1),
            mb, wd,
            jnp.concatenate(
                [row(p['mp1_b2'][i]), row(p['mp2_b2'][i])], axis=1),
            (aw_[0] - aw_[1]).reshape(2 * H, 1),
            (ab_[0] - ab_[1]).reshape(1, 1),
            p['scale_factor'][i].reshape(1, 1).astype(f32),
        )
        ag = _sc_scatter(msg, dst2d_s, zeros_pad)
        gw = p['gate_w'][i].astype(f32)
        u1w = p['upd1_w'][i].astype(f32)
        h, acc, hsum = _upd_call(
            kind, i % 2 == 1, ag, h, acc,
            gw[:, :H].T, gw[:, H:].T, row(p['gate_b'][i]),
            u1w[:, :H].T, u1w[:, H:].T, row(p['upd1_b'][i]),
            p['upd2_w'][i].T.astype(f32), row(p['upd2_b'][i]),
            attn[i].reshape(1, 1),
        )

    return _head_call(
        hsum,
        p['pre_w1'].T.astype(f32), row(p['pre_b1']),
        p['pre_w2'].T.astype(f32), row(p['pre_b2']),
        p['out_w'].T.astype(f32), row(p['out_b']),
    )
